# speedup vs baseline: 3.7370x; 3.7370x over previous
import functools

import jax
import jax.numpy as jnp
from jax import lax
from jax.experimental import pallas as pl
from jax.experimental.pallas import tpu as pltpu
from jax.experimental.pallas import tpu_sc as plsc

_NC = 2
_NS = 16
_NW = _NC * _NS
_CHUNK = 64


@functools.partial(jax.jit, static_argnums=(0, 1, 2, 3))
def _build_and_run(batch, seq_len, embed, nchunk, table, pos):
    mesh = plsc.VectorSubcoreMesh(core_axis_name="c", subcore_axis_name="s")

    @functools.partial(
        pl.kernel,
        out_type=jax.ShapeDtypeStruct((batch * seq_len, embed), jnp.float32),
        mesh=mesh,
        scratch_types=[
            pltpu.VMEM((nchunk, _CHUNK), jnp.int32),
            pltpu.VMEM((_CHUNK, embed), jnp.float32),
            pltpu.VMEM((_CHUNK, embed), jnp.float32),
            pltpu.SemaphoreType.DMA,
            pltpu.SemaphoreType.DMA,
            pltpu.SemaphoreType.DMA,
            pltpu.SemaphoreType.DMA,
        ],
    )
    def pos_embed(table_hbm, pos_hbm, out_hbm, idx_v, buf0, buf1, g0, g1, w0, w1):
        wid = lax.axis_index("s") * _NC + lax.axis_index("c")
        rpw = nchunk * _CHUNK
        base = wid * rpw
        pltpu.sync_copy(pos_hbm.at[wid], idx_v)
        bufs = (buf0, buf1)
        gsems = (g0, g1)
        wsems = (w0, w1)
        gh = [None] * nchunk
        wh = [[] for _ in range(nchunk)]
        gh[0] = pltpu.async_copy(table_hbm.at[idx_v.at[0]], bufs[0], gsems[0])
        for i in range(nchunk):
            sl = i % 2
            if i >= 1:
                for h in wh[i - 1]:
                    h.wait()
            if i + 1 < nchunk:
                gh[i + 1] = pltpu.async_copy(
                    table_hbm.at[idx_v.at[i + 1]], bufs[1 - sl], gsems[1 - sl])
            gh[i].wait()
            for b in range(batch):
                wh[i].append(pltpu.async_copy(
                    bufs[sl],
                    out_hbm.at[pl.ds(b * seq_len + base + i * _CHUNK, _CHUNK)],
                    wsems[sl]))
        for h in wh[nchunk - 1]:
            h.wait()

    return pos_embed(table, pos)


def kernel(inputs, length, table):
    batch, seq_len = inputs.shape
    vocab, embed = table.shape
    pos = jnp.clip(
        jnp.arange(seq_len, dtype=jnp.int32) + jnp.asarray(length, jnp.int32),
        0, vocab - 1)
    nchunk = seq_len // _NW // _CHUNK
    pos = pos.reshape(_NW, nchunk, _CHUNK)
    out = _build_and_run(batch, seq_len, embed, nchunk, table, pos)
    return out.reshape(batch, seq_len, embed)
